# baseline (speedup 1.0000x reference)
import functools

import jax
import jax.numpy as jnp
from jax import lax
from jax.experimental import pallas as pl
from jax.experimental.pallas import tpu as pltpu
from jax.experimental.pallas import tpu_sc as plsc

N_PTS = 50000
C = 128
K = 27
M = 20000

E_TOT = K * M
BLK = 2048
NBLK = -(-E_TOT // BLK)
E_PAD = NBLK * BLK

NCHUNK = 4
R = 12512
TRASH = R
ACC_ROWS = R + 8
U_ROWS = 544
UNITS = R // U_ROWS
LAST_U = N_PTS - 3 * R - (UNITS - 1) * U_ROWS
DEAD = BLK

_TCB = 5000


def _tc_body(x_ref, w_ref, y_ref):
    y_ref[...] = jnp.dot(x_ref[...], w_ref[0],
                         preferred_element_type=jnp.float32)


def _compute_y(in_feat, w):
    nb = N_PTS // _TCB
    return pl.pallas_call(
        _tc_body,
        grid=(nb, K),
        in_specs=[
            pl.BlockSpec((_TCB, C), lambda i, k: (i, 0)),
            pl.BlockSpec((1, C, C), lambda i, k: (k, 0, 0)),
        ],
        out_specs=pl.BlockSpec((_TCB, C), lambda i, k: (k * nb + i, 0)),
        out_shape=jax.ShapeDtypeStruct((K * N_PTS, C), jnp.float32),
    )(in_feat, w)


def _sc_body(y_hbm, gi_hbm, oi_hbm, z_hbm, out_hbm,
             acc, gi_v, oi_v, cg_v, co_v, rows_v):
    c = lax.axis_index("c")
    s = lax.axis_index("s")
    lanes = lax.iota(jnp.int32, 16)

    for p in range(2):
        chunk = c * 2 + p
        lo = chunk * R

        for ui in range(2):
            u = s + ui * 16

            @pl.when(u < UNITS)
            def _():
                pltpu.sync_copy(z_hbm.at[pl.ds(u * U_ROWS, U_ROWS)],
                                acc.at[pl.ds(u * U_ROWS, U_ROWS)])
        plsc.subcore_barrier()

        def blk_body(i, carry):
            b = s + i * 16

            @pl.when(b < NBLK)
            def _():
                pltpu.sync_copy(gi_hbm.at[b], gi_v)
                pltpu.sync_copy(oi_hbm.at[b], oi_v)
                off = jnp.int32(0)
                for t in range(16):
                    def grp(l, off):
                        ov = oi_v[t, pl.ds(l * 16, 16)]
                        gv = gi_v[t, pl.ds(l * 16, 16)]
                        m = (ov >= lo) & (ov < lo + R)
                        mi = m.astype(jnp.int32)
                        x = mi
                        for sh in (1, 2, 4, 8):
                            g = x.at[jnp.maximum(lanes - sh, 0)].get(
                                mode="promise_in_bounds")
                            x = x + jnp.where(lanes >= sh, g, 0)
                        pos = jnp.where(m, off + x - mi, DEAD)
                        plsc.store_scatter(cg_v, [pos >> 7, pos & 127], gv)
                        plsc.store_scatter(co_v, [pos >> 7, pos & 127],
                                           ov - lo)
                        return off + x[15]
                    off = lax.fori_loop(0, 8, grp, off)
                nstream = (off + 127) // 128
                tr = jnp.full((16,), TRASH, jnp.int32)
                zr = jnp.zeros((16,), jnp.int32)
                for j in range(8):
                    pos = off + j * 16 + lanes
                    pos = jnp.where(pos < nstream * 128, pos, DEAD)
                    plsc.store_scatter(co_v, [pos >> 7, pos & 127], tr)
                    plsc.store_scatter(cg_v, [pos >> 7, pos & 127], zr)

                for si in range(16):
                    @pl.when(si < nstream)
                    def _():
                        pltpu.sync_copy(y_hbm.at[cg_v.at[si]], rows_v)
                        pltpu.sync_copy(rows_v, acc.at[co_v.at[si]],
                                        add=True)
            return carry

        lax.fori_loop(0, -(-NBLK // 16), blk_body, 0)
        plsc.subcore_barrier()

        for ui in range(2):
            u = s + ui * 16

            @pl.when((u < UNITS - 1) | ((u == UNITS - 1) & (chunk < 3)))
            def _():
                pltpu.sync_copy(
                    acc.at[pl.ds(u * U_ROWS, U_ROWS)],
                    out_hbm.at[pl.ds(chunk * R + u * U_ROWS, U_ROWS)])

            @pl.when((u == UNITS - 1) & (chunk == 3))
            def _():
                pltpu.sync_copy(
                    acc.at[pl.ds(u * U_ROWS, LAST_U)],
                    out_hbm.at[pl.ds(chunk * R + u * U_ROWS, LAST_U)])
        plsc.subcore_barrier()


@functools.partial(
    pl.kernel,
    out_type=jax.ShapeDtypeStruct((N_PTS, C), jnp.float32),
    mesh=plsc.VectorSubcoreMesh(core_axis_name="c", subcore_axis_name="s"),
    compiler_params=pltpu.CompilerParams(needs_layout_passes=False),
    scratch_types=[
        pltpu.VMEM_SHARED((ACC_ROWS, C), jnp.float32),
        pltpu.VMEM((16, BLK // 16), jnp.int32),
        pltpu.VMEM((16, BLK // 16), jnp.int32),
        pltpu.VMEM((17, BLK // 16), jnp.int32),
        pltpu.VMEM((17, BLK // 16), jnp.int32),
        pltpu.VMEM((BLK // 16, C), jnp.float32),
    ],
)
def _sc_scatter(y_hbm, gi_hbm, oi_hbm, z_hbm, out_hbm,
                acc, gi_v, oi_v, cg_v, co_v, rows_v):
    _sc_body(y_hbm, gi_hbm, oi_hbm, z_hbm, out_hbm,
             acc, gi_v, oi_v, cg_v, co_v, rows_v)


def kernel(kernel, in_feat, in_maps, out_maps):
    w = kernel
    y = _compute_y(in_feat, w)

    k_off = (jnp.arange(K, dtype=jnp.int32) * N_PTS)[:, None]
    gidx = (in_maps + k_off).reshape(-1)
    oidx = out_maps.reshape(-1)
    pad = E_PAD - E_TOT
    gidx = jnp.concatenate(
        [gidx, jnp.zeros((pad,), jnp.int32)]).reshape(NBLK, 16, BLK // 16)
    oidx = jnp.concatenate(
        [oidx, jnp.full((pad,), jnp.int32(1 << 30))]
    ).reshape(NBLK, 16, BLK // 16)
    zeros = jnp.zeros((R, C), jnp.float32)

    return _sc_scatter(y, gidx, oidx, zeros)


# speedup vs baseline: 1.0002x; 1.0002x over previous
import functools

import jax
import jax.numpy as jnp
from jax import lax
from jax.experimental import pallas as pl
from jax.experimental.pallas import tpu as pltpu
from jax.experimental.pallas import tpu_sc as plsc

N_PTS = 50000
C = 128
K = 27
M = 20000

E_TOT = K * M
BLK = 2048
NBLK = -(-E_TOT // BLK)
E_PAD = NBLK * BLK

NCHUNK = 4
R = 12512
TRASH = R
ACC_ROWS = R + 8
U_ROWS = 544
UNITS = R // U_ROWS
LAST_U = N_PTS - 3 * R - (UNITS - 1) * U_ROWS
DEAD = BLK + 16

_TCB = 5000


def _tc_body(x_ref, w_ref, y_ref):
    y_ref[...] = jnp.dot(x_ref[...], w_ref[0],
                         preferred_element_type=jnp.float32)


def _compute_y(in_feat, w):
    nb = N_PTS // _TCB
    return pl.pallas_call(
        _tc_body,
        grid=(nb, K),
        in_specs=[
            pl.BlockSpec((_TCB, C), lambda i, k: (i, 0)),
            pl.BlockSpec((1, C, C), lambda i, k: (k, 0, 0)),
        ],
        out_specs=pl.BlockSpec((_TCB, C), lambda i, k: (k * nb + i, 0)),
        out_shape=jax.ShapeDtypeStruct((K * N_PTS, C), jnp.float32),
    )(in_feat, w)


def _sc_body(y_hbm, gi_hbm, oi_hbm, z_hbm, out_hbm,
             acc, gi_v, oi_v, cg_v, co_v, rows_v):
    c = lax.axis_index("c")
    s = lax.axis_index("s")
    lanes = lax.iota(jnp.int32, 16)

    for p in range(2):
        chunk = c * 2 + p
        lo = chunk * R

        for ui in range(2):
            u = s + ui * 16

            @pl.when(u < UNITS)
            def _():
                pltpu.sync_copy(z_hbm.at[pl.ds(u * U_ROWS, U_ROWS)],
                                acc.at[pl.ds(u * U_ROWS, U_ROWS)])
        plsc.subcore_barrier()

        def blk_body(i, carry):
            b = s + i * 16

            @pl.when(b < NBLK)
            def _():
                pltpu.sync_copy(gi_hbm.at[b], gi_v)
                pltpu.sync_copy(oi_hbm.at[b], oi_v)
                off = jnp.int32(0)
                for t in range(16):
                    def grp(l, off):
                        ov = oi_v[t, pl.ds(l * 16, 16)]
                        gv = gi_v[t, pl.ds(l * 16, 16)]
                        m = (ov >= lo) & (ov < lo + R)
                        key = 1 - m.astype(jnp.int32)
                        _, sgv = plsc.sort_key_val(key, gv)
                        _, sov = plsc.sort_key_val(key, ov - lo)
                        cg_v[pl.ds(off, 16)] = sgv
                        co_v[pl.ds(off, 16)] = sov
                        return off + plsc.all_reduce_population_count(m)[0]
                    off = lax.fori_loop(0, 8, grp, off)
                nstream = (off + 127) // 128
                tr = jnp.full((16,), TRASH, jnp.int32)
                zr = jnp.zeros((16,), jnp.int32)
                for j in range(8):
                    pos = off + j * 16 + lanes
                    pos = jnp.where(pos < nstream * 128, pos, DEAD)
                    plsc.store_scatter(co_v, [pos], tr)
                    plsc.store_scatter(cg_v, [pos], zr)

                for si in range(16):
                    @pl.when(si < nstream)
                    def _():
                        pltpu.sync_copy(
                            y_hbm.at[cg_v.at[pl.ds(si * 128, 128)]], rows_v)
                        pltpu.sync_copy(
                            rows_v, acc.at[co_v.at[pl.ds(si * 128, 128)]],
                            add=True)
            return carry

        lax.fori_loop(0, -(-NBLK // 16), blk_body, 0)
        plsc.subcore_barrier()

        for ui in range(2):
            u = s + ui * 16

            @pl.when((u < UNITS - 1) | ((u == UNITS - 1) & (chunk < 3)))
            def _():
                pltpu.sync_copy(
                    acc.at[pl.ds(u * U_ROWS, U_ROWS)],
                    out_hbm.at[pl.ds(chunk * R + u * U_ROWS, U_ROWS)])

            @pl.when((u == UNITS - 1) & (chunk == 3))
            def _():
                pltpu.sync_copy(
                    acc.at[pl.ds(u * U_ROWS, LAST_U)],
                    out_hbm.at[pl.ds(chunk * R + u * U_ROWS, LAST_U)])
        plsc.subcore_barrier()


@functools.partial(
    pl.kernel,
    out_type=jax.ShapeDtypeStruct((N_PTS, C), jnp.float32),
    mesh=plsc.VectorSubcoreMesh(core_axis_name="c", subcore_axis_name="s"),
    compiler_params=pltpu.CompilerParams(needs_layout_passes=False),
    scratch_types=[
        pltpu.VMEM_SHARED((ACC_ROWS, C), jnp.float32),
        pltpu.VMEM((16, BLK // 16), jnp.int32),
        pltpu.VMEM((16, BLK // 16), jnp.int32),
        pltpu.VMEM((BLK + 32,), jnp.int32),
        pltpu.VMEM((BLK + 32,), jnp.int32),
        pltpu.VMEM((BLK // 16, C), jnp.float32),
    ],
)
def _sc_scatter(y_hbm, gi_hbm, oi_hbm, z_hbm, out_hbm,
                acc, gi_v, oi_v, cg_v, co_v, rows_v):
    _sc_body(y_hbm, gi_hbm, oi_hbm, z_hbm, out_hbm,
             acc, gi_v, oi_v, cg_v, co_v, rows_v)


def kernel(kernel, in_feat, in_maps, out_maps):
    w = kernel
    y = _compute_y(in_feat, w)

    k_off = (jnp.arange(K, dtype=jnp.int32) * N_PTS)[:, None]
    gidx = (in_maps + k_off).reshape(-1)
    oidx = out_maps.reshape(-1)
    pad = E_PAD - E_TOT
    gidx = jnp.concatenate(
        [gidx, jnp.zeros((pad,), jnp.int32)]).reshape(NBLK, 16, BLK // 16)
    oidx = jnp.concatenate(
        [oidx, jnp.full((pad,), jnp.int32(1 << 30))]
    ).reshape(NBLK, 16, BLK // 16)
    zeros = jnp.zeros((R, C), jnp.float32)

    return _sc_scatter(y, gidx, oidx, zeros)


# speedup vs baseline: 1.8219x; 1.8216x over previous
import functools

import jax
import jax.numpy as jnp
from jax import lax
from jax.experimental import pallas as pl
from jax.experimental.pallas import tpu as pltpu
from jax.experimental.pallas import tpu_sc as plsc

N_PTS = 50000
C = 128
K = 27
M = 20000

E_TOT = K * M
BLK = 2048
NBLK = -(-E_TOT // BLK)
E_PAD = NBLK * BLK

NCHUNK = 4
R = 12512
TRASH = R
ACC_ROWS = R + 8
U_ROWS = 544
UNITS = R // U_ROWS
LAST_U = N_PTS - 3 * R - (UNITS - 1) * U_ROWS

_TCB = 5000


def _tc_body(x_ref, w_ref, y_ref):
    y_ref[...] = jnp.dot(x_ref[...], w_ref[0],
                         preferred_element_type=jnp.float32)


def _compute_y(in_feat, w):
    nb = N_PTS // _TCB
    return pl.pallas_call(
        _tc_body,
        grid=(nb, K),
        in_specs=[
            pl.BlockSpec((_TCB, C), lambda i, k: (i, 0)),
            pl.BlockSpec((1, C, C), lambda i, k: (k, 0, 0)),
        ],
        out_specs=pl.BlockSpec((_TCB, C), lambda i, k: (k * nb + i, 0)),
        out_shape=jax.ShapeDtypeStruct((K * N_PTS, C), jnp.float32),
    )(in_feat, w)


def _sc_body(y_hbm, gi_hbm, oi_hbm, z_hbm, out_hbm,
             acc, gi_v, oi_v, co_v, rows_v):
    c = lax.axis_index("c")
    s = lax.axis_index("s")

    for p in range(2):
        chunk = c * 2 + p
        lo = chunk * R

        for ui in range(2):
            u = s + ui * 16

            @pl.when(u < UNITS)
            def _():
                pltpu.sync_copy(z_hbm.at[pl.ds(u * U_ROWS, U_ROWS)],
                                acc.at[pl.ds(u * U_ROWS, U_ROWS)])
        plsc.subcore_barrier()

        def blk_body(i, carry):
            b = s + i * 16

            @pl.when(b < NBLK)
            def _():
                pltpu.sync_copy(gi_hbm.at[b], gi_v)
                pltpu.sync_copy(oi_hbm.at[b], oi_v)
                for t in range(16):
                    def grp(l, _):
                        ov = oi_v[t, pl.ds(l * 16, 16)]
                        m = (ov >= lo) & (ov < lo + R)
                        co_v[t, pl.ds(l * 16, 16)] = jnp.where(
                            m, ov - lo, TRASH)
                        return 0
                    lax.fori_loop(0, 8, grp, 0)
                    pltpu.sync_copy(y_hbm.at[gi_v.at[t]], rows_v)
                    pltpu.sync_copy(rows_v, acc.at[co_v.at[t]], add=True)
            return carry

        lax.fori_loop(0, -(-NBLK // 16), blk_body, 0)
        plsc.subcore_barrier()

        for ui in range(2):
            u = s + ui * 16

            @pl.when((u < UNITS - 1) | ((u == UNITS - 1) & (chunk < 3)))
            def _():
                pltpu.sync_copy(
                    acc.at[pl.ds(u * U_ROWS, U_ROWS)],
                    out_hbm.at[pl.ds(chunk * R + u * U_ROWS, U_ROWS)])

            @pl.when((u == UNITS - 1) & (chunk == 3))
            def _():
                pltpu.sync_copy(
                    acc.at[pl.ds(u * U_ROWS, LAST_U)],
                    out_hbm.at[pl.ds(chunk * R + u * U_ROWS, LAST_U)])
        plsc.subcore_barrier()


@functools.partial(
    pl.kernel,
    out_type=jax.ShapeDtypeStruct((N_PTS, C), jnp.float32),
    mesh=plsc.VectorSubcoreMesh(core_axis_name="c", subcore_axis_name="s"),
    scratch_types=[
        pltpu.VMEM_SHARED((ACC_ROWS, C), jnp.float32),
        pltpu.VMEM((16, BLK // 16), jnp.int32),
        pltpu.VMEM((16, BLK // 16), jnp.int32),
        pltpu.VMEM((16, BLK // 16), jnp.int32),
        pltpu.VMEM((BLK // 16, C), jnp.float32),
    ],
)
def _sc_scatter(y_hbm, gi_hbm, oi_hbm, z_hbm, out_hbm,
                acc, gi_v, oi_v, co_v, rows_v):
    _sc_body(y_hbm, gi_hbm, oi_hbm, z_hbm, out_hbm,
             acc, gi_v, oi_v, co_v, rows_v)


def kernel(kernel, in_feat, in_maps, out_maps):
    w = kernel
    y = _compute_y(in_feat, w)

    k_off = (jnp.arange(K, dtype=jnp.int32) * N_PTS)[:, None]
    gidx = (in_maps + k_off).reshape(-1)
    oidx = out_maps.reshape(-1)
    pad = E_PAD - E_TOT
    gidx = jnp.concatenate(
        [gidx, jnp.zeros((pad,), jnp.int32)]).reshape(NBLK, 16, BLK // 16)
    oidx = jnp.concatenate(
        [oidx, jnp.full((pad,), jnp.int32(1 << 30))]
    ).reshape(NBLK, 16, BLK // 16)
    zeros = jnp.zeros((R, C), jnp.float32)

    return _sc_scatter(y, gidx, oidx, zeros)
